# baseline (speedup 1.0000x reference)
import functools

import jax
import jax.numpy as jnp
from jax import lax
from jax.experimental import pallas as pl
from jax.experimental.pallas import tpu as pltpu
from jax.experimental.pallas import tpu_sc as plsc

_N = 10000
_E = 320000
_D = 128
_DH = 64
_NCLASS = 16

_NCH = 316
_K = 64
_EPAD = 16 * _NCH * _K
_NPAD = 10112
_ROWS_PER_TILE = _NPAD // 16

_BN = 1000


def _sc_scatter_build():
    mesh = plsc.VectorSubcoreMesh(core_axis_name="c", subcore_axis_name="s")

    @functools.partial(
        pl.kernel,
        out_type=jax.ShapeDtypeStruct((2, _NPAD, _DH), jnp.float32),
        mesh=mesh,
        scratch_types=[
            pltpu.VMEM((_NCH, _K), jnp.int32),
            pltpu.VMEM((_NCH, _K), jnp.int32),
            pltpu.VMEM((2, _K, _DH), jnp.float32),
            pltpu.VMEM((_ROWS_PER_TILE, _DH), jnp.float32),
            pltpu.VMEM_SHARED((_NPAD, _DH), jnp.float32),
            [pltpu.SemaphoreType.DMA] * 2,
        ],
        compiler_params=pltpu.CompilerParams(use_tc_tiling_on_sc=False),
    )
    def sc_scatter(m2_hbm, src_hbm, dst_hbm, out_hbm,
                   src_v, dst_v, rows_v, z_v, acc_sh, gsems):
        c = lax.axis_index("c")
        s = lax.axis_index("s")

        pltpu.sync_copy(src_hbm.at[s], src_v)
        pltpu.sync_copy(dst_hbm.at[s], dst_v)

        zero = jnp.zeros((16,), jnp.float32)

        def zrow(r, carry):
            for cc in range(_DH // 16):
                z_v[r, pl.ds(cc * 16, 16)] = zero
            return carry

        lax.fori_loop(0, _ROWS_PER_TILE, zrow, 0)

        row0 = s * _ROWS_PER_TILE
        pltpu.sync_copy(z_v, acc_sh.at[pl.ds(row0, _ROWS_PER_TILE)])
        plsc.subcore_barrier()

        m_ref = m2_hbm.at[c]

        def body(jj, carry):
            j0 = 4 * jj
            cps = [None, None]
            cps[0] = pltpu.async_copy(
                m_ref.at[src_v.at[j0]], rows_v.at[0], gsems[0])
            for u in range(4):
                cps[u % 2].wait()
                if u + 1 < 4:
                    nu = (u + 1) % 2
                    cps[nu] = pltpu.async_copy(
                        m_ref.at[src_v.at[j0 + u + 1]], rows_v.at[nu],
                        gsems[nu])
                pltpu.sync_copy(
                    rows_v.at[u % 2], acc_sh.at[dst_v.at[j0 + u]], add=True)
            return carry

        lax.fori_loop(0, _NCH // 4, body, 0)

        plsc.subcore_barrier()

        pltpu.sync_copy(acc_sh.at[pl.ds(row0, _ROWS_PER_TILE)],
                        out_hbm.at[c, pl.ds(row0, _ROWS_PER_TILE)])

    return sc_scatter


_sc_scatter = _sc_scatter_build()


def _edge_mm_body(h_ref, W_ref, b_ref, m2_ref):
    m = (
        jnp.dot(h_ref[...], W_ref[...], preferred_element_type=jnp.float32)
        + b_ref[...]
    )
    m2_ref[0] = m[:, :_DH]
    m2_ref[1] = m[:, _DH:]


def _gru(p_ref, h_ref, W_ih_ref, W_hh_ref, b_ih_ref, b_hh_ref):
    a_lo = p_ref[0]
    a_hi = p_ref[1]
    gi = (
        jnp.dot(a_lo, W_ih_ref[...][:_DH, :], preferred_element_type=jnp.float32)
        + jnp.dot(a_hi, W_ih_ref[...][_DH:, :], preferred_element_type=jnp.float32)
        + b_ih_ref[...]
    )
    h = h_ref[...]
    gh = jnp.dot(h, W_hh_ref[...], preferred_element_type=jnp.float32) + b_hh_ref[...]
    r = jax.nn.sigmoid(gi[:, :_D] + gh[:, :_D])
    z = jax.nn.sigmoid(gi[:, _D:2 * _D] + gh[:, _D:2 * _D])
    n = jnp.tanh(gi[:, 2 * _D:] + r * gh[:, 2 * _D:])
    return (1.0 - z) * n + z * h


def _gru_edge_body(p_ref, h_ref, W_ih_ref, W_hh_ref, b_ih_ref, b_hh_ref,
                   W_edge_ref, b_edge_ref, hn_ref, m2_ref):
    hn = _gru(p_ref, h_ref, W_ih_ref, W_hh_ref, b_ih_ref, b_hh_ref)
    hn_ref[...] = hn
    m = (
        jnp.dot(hn, W_edge_ref[...], preferred_element_type=jnp.float32)
        + b_edge_ref[...]
    )
    m2_ref[0] = m[:, :_DH]
    m2_ref[1] = m[:, _DH:]


def _gru_fc_body(p_ref, h_ref, W_ih_ref, W_hh_ref, b_ih_ref, b_hh_ref,
                 W_fc_ref, b_fc_ref, out_ref):
    hn = _gru(p_ref, h_ref, W_ih_ref, W_hh_ref, b_ih_ref, b_hh_ref)
    e = jnp.where(hn > 0, hn, jnp.exp(jnp.minimum(hn, 0.0)) - 1.0)
    out_ref[...] = (
        jnp.dot(e, W_fc_ref[...], preferred_element_type=jnp.float32)
        + b_fc_ref[...]
    )


def _full(shape):
    return pl.BlockSpec(shape, lambda i: tuple(0 for _ in shape))


_GRID = _N // _BN

_edge_mm = pl.pallas_call(
    _edge_mm_body,
    grid=(_GRID,),
    in_specs=[
        pl.BlockSpec((_BN, _D), lambda i: (i, 0)),
        _full((_D, _D)),
        _full((1, _D)),
    ],
    out_specs=pl.BlockSpec((2, _BN, _DH), lambda i: (0, i, 0)),
    out_shape=jax.ShapeDtypeStruct((2, _N, _DH), jnp.float32),
)

_gru_edge = pl.pallas_call(
    _gru_edge_body,
    grid=(_GRID,),
    in_specs=[
        pl.BlockSpec((2, _BN, _DH), lambda i: (0, i, 0)),
        pl.BlockSpec((_BN, _D), lambda i: (i, 0)),
        _full((_D, 3 * _D)),
        _full((_D, 3 * _D)),
        _full((1, 3 * _D)),
        _full((1, 3 * _D)),
        _full((_D, _D)),
        _full((1, _D)),
    ],
    out_specs=[
        pl.BlockSpec((_BN, _D), lambda i: (i, 0)),
        pl.BlockSpec((2, _BN, _DH), lambda i: (0, i, 0)),
    ],
    out_shape=[
        jax.ShapeDtypeStruct((_N, _D), jnp.float32),
        jax.ShapeDtypeStruct((2, _N, _DH), jnp.float32),
    ],
)

_gru_fc = pl.pallas_call(
    _gru_fc_body,
    grid=(_GRID,),
    in_specs=[
        pl.BlockSpec((2, _BN, _DH), lambda i: (0, i, 0)),
        pl.BlockSpec((_BN, _D), lambda i: (i, 0)),
        _full((_D, 3 * _D)),
        _full((_D, 3 * _D)),
        _full((1, 3 * _D)),
        _full((1, 3 * _D)),
        _full((_D, _NCLASS)),
        _full((1, _NCLASS)),
    ],
    out_specs=pl.BlockSpec((_BN, _NCLASS), lambda i: (i, 0)),
    out_shape=jax.ShapeDtypeStruct((_N, _NCLASS), jnp.float32),
)


def kernel(x, edge_index, W_edge, b_edge, W_ih, W_hh, b_ih, b_hh, W_fc, b_fc):
    npad = _EPAD - _E
    src = jnp.concatenate(
        [edge_index[0].astype(jnp.int32), jnp.zeros((npad,), jnp.int32)]
    ).reshape(16, _NCH, _K)
    dst = jnp.concatenate(
        [edge_index[1].astype(jnp.int32),
         10104 + (jnp.arange(npad, dtype=jnp.int32) & 7)]
    ).reshape(16, _NCH, _K)
    b_edge2 = b_edge.reshape(1, _D)
    b_ih2 = b_ih.reshape(1, 3 * _D)
    b_hh2 = b_hh.reshape(1, 3 * _D)
    b_fc2 = b_fc.reshape(1, _NCLASS)

    m1 = _edge_mm(x, W_edge, b_edge2)
    p1 = _sc_scatter(m1, src, dst)
    h1, m2 = _gru_edge(p1, x, W_ih, W_hh, b_ih2, b_hh2, W_edge, b_edge2)
    p2 = _sc_scatter(m2, src, dst)
    logits = _gru_fc(p2, h1, W_ih, W_hh, b_ih2, b_hh2, W_fc, b_fc2)
    return logits


# speedup vs baseline: 1.3134x; 1.3134x over previous
import functools

import jax
import jax.numpy as jnp
from jax import lax
from jax.experimental import pallas as pl
from jax.experimental.pallas import tpu as pltpu
from jax.experimental.pallas import tpu_sc as plsc

_N = 10000
_E = 320000
_D = 128
_DH = 64
_NCLASS = 16

_NCH = 157
_K = 128
_EPAD = 16 * _NCH * _K
_NPAD = 10112
_ROWS_PER_TILE = _NPAD // 16

_BN = 1000


def _sc_scatter_build():
    mesh = plsc.VectorSubcoreMesh(core_axis_name="c", subcore_axis_name="s")

    @functools.partial(
        pl.kernel,
        out_type=jax.ShapeDtypeStruct((2, _NPAD, _DH), jnp.float32),
        mesh=mesh,
        scratch_types=[
            pltpu.VMEM((_NCH, _K), jnp.int32),
            pltpu.VMEM((_NCH, _K), jnp.int32),
            pltpu.VMEM((_K, _DH), jnp.float32),
            pltpu.VMEM((_ROWS_PER_TILE, _DH), jnp.float32),
            pltpu.VMEM_SHARED((_NPAD, _DH), jnp.float32),
            pltpu.SemaphoreType.DMA,
        ],
        compiler_params=pltpu.CompilerParams(use_tc_tiling_on_sc=False),
    )
    def sc_scatter(m2_hbm, src_hbm, dst_hbm, out_hbm,
                   src_v, dst_v, rows_v, z_v, acc_sh, gsem):
        c = lax.axis_index("c")
        s = lax.axis_index("s")

        pltpu.sync_copy(src_hbm.at[s], src_v)
        pltpu.sync_copy(dst_hbm.at[s], dst_v)

        zero = jnp.zeros((16,), jnp.float32)

        def zrow(r, carry):
            for cc in range(_DH // 16):
                z_v[r, pl.ds(cc * 16, 16)] = zero
            return carry

        lax.fori_loop(0, _ROWS_PER_TILE, zrow, 0)

        row0 = s * _ROWS_PER_TILE
        pltpu.sync_copy(z_v, acc_sh.at[pl.ds(row0, _ROWS_PER_TILE)])
        plsc.subcore_barrier()

        m_ref = m2_hbm.at[c]

        def body(j, carry):
            pltpu.async_copy(m_ref.at[src_v.at[j]], rows_v, gsem).wait()
            pltpu.sync_copy(rows_v, acc_sh.at[dst_v.at[j]], add=True)
            return carry

        lax.fori_loop(0, _NCH, body, 0)

        plsc.subcore_barrier()

        pltpu.sync_copy(acc_sh.at[pl.ds(row0, _ROWS_PER_TILE)],
                        out_hbm.at[c, pl.ds(row0, _ROWS_PER_TILE)])

    return sc_scatter


_sc_scatter = _sc_scatter_build()


def _edge_mm_body(h_ref, W_ref, b_ref, m2_ref):
    m = (
        jnp.dot(h_ref[...], W_ref[...], preferred_element_type=jnp.float32)
        + b_ref[...]
    )
    m2_ref[0] = m[:, :_DH]
    m2_ref[1] = m[:, _DH:]


def _gru(p_ref, h_ref, W_ih_ref, W_hh_ref, b_ih_ref, b_hh_ref):
    a_lo = p_ref[0]
    a_hi = p_ref[1]
    gi = (
        jnp.dot(a_lo, W_ih_ref[...][:_DH, :], preferred_element_type=jnp.float32)
        + jnp.dot(a_hi, W_ih_ref[...][_DH:, :], preferred_element_type=jnp.float32)
        + b_ih_ref[...]
    )
    h = h_ref[...]
    gh = jnp.dot(h, W_hh_ref[...], preferred_element_type=jnp.float32) + b_hh_ref[...]
    r = jax.nn.sigmoid(gi[:, :_D] + gh[:, :_D])
    z = jax.nn.sigmoid(gi[:, _D:2 * _D] + gh[:, _D:2 * _D])
    n = jnp.tanh(gi[:, 2 * _D:] + r * gh[:, 2 * _D:])
    return (1.0 - z) * n + z * h


def _gru_edge_body(p_ref, h_ref, W_ih_ref, W_hh_ref, b_ih_ref, b_hh_ref,
                   W_edge_ref, b_edge_ref, hn_ref, m2_ref):
    hn = _gru(p_ref, h_ref, W_ih_ref, W_hh_ref, b_ih_ref, b_hh_ref)
    hn_ref[...] = hn
    m = (
        jnp.dot(hn, W_edge_ref[...], preferred_element_type=jnp.float32)
        + b_edge_ref[...]
    )
    m2_ref[0] = m[:, :_DH]
    m2_ref[1] = m[:, _DH:]


def _gru_fc_body(p_ref, h_ref, W_ih_ref, W_hh_ref, b_ih_ref, b_hh_ref,
                 W_fc_ref, b_fc_ref, out_ref):
    hn = _gru(p_ref, h_ref, W_ih_ref, W_hh_ref, b_ih_ref, b_hh_ref)
    e = jnp.where(hn > 0, hn, jnp.exp(jnp.minimum(hn, 0.0)) - 1.0)
    out_ref[...] = (
        jnp.dot(e, W_fc_ref[...], preferred_element_type=jnp.float32)
        + b_fc_ref[...]
    )


def _full(shape):
    return pl.BlockSpec(shape, lambda i: tuple(0 for _ in shape))


_GRID = _N // _BN

_edge_mm = pl.pallas_call(
    _edge_mm_body,
    grid=(_GRID,),
    in_specs=[
        pl.BlockSpec((_BN, _D), lambda i: (i, 0)),
        _full((_D, _D)),
        _full((1, _D)),
    ],
    out_specs=pl.BlockSpec((2, _BN, _DH), lambda i: (0, i, 0)),
    out_shape=jax.ShapeDtypeStruct((2, _N, _DH), jnp.float32),
)

_gru_edge = pl.pallas_call(
    _gru_edge_body,
    grid=(_GRID,),
    in_specs=[
        pl.BlockSpec((2, _BN, _DH), lambda i: (0, i, 0)),
        pl.BlockSpec((_BN, _D), lambda i: (i, 0)),
        _full((_D, 3 * _D)),
        _full((_D, 3 * _D)),
        _full((1, 3 * _D)),
        _full((1, 3 * _D)),
        _full((_D, _D)),
        _full((1, _D)),
    ],
    out_specs=[
        pl.BlockSpec((_BN, _D), lambda i: (i, 0)),
        pl.BlockSpec((2, _BN, _DH), lambda i: (0, i, 0)),
    ],
    out_shape=[
        jax.ShapeDtypeStruct((_N, _D), jnp.float32),
        jax.ShapeDtypeStruct((2, _N, _DH), jnp.float32),
    ],
)

_gru_fc = pl.pallas_call(
    _gru_fc_body,
    grid=(_GRID,),
    in_specs=[
        pl.BlockSpec((2, _BN, _DH), lambda i: (0, i, 0)),
        pl.BlockSpec((_BN, _D), lambda i: (i, 0)),
        _full((_D, 3 * _D)),
        _full((_D, 3 * _D)),
        _full((1, 3 * _D)),
        _full((1, 3 * _D)),
        _full((_D, _NCLASS)),
        _full((1, _NCLASS)),
    ],
    out_specs=pl.BlockSpec((_BN, _NCLASS), lambda i: (i, 0)),
    out_shape=jax.ShapeDtypeStruct((_N, _NCLASS), jnp.float32),
)


def kernel(x, edge_index, W_edge, b_edge, W_ih, W_hh, b_ih, b_hh, W_fc, b_fc):
    npad = _EPAD - _E
    src = jnp.concatenate(
        [edge_index[0].astype(jnp.int32), jnp.zeros((npad,), jnp.int32)]
    ).reshape(16, _NCH, _K)
    dst = jnp.concatenate(
        [edge_index[1].astype(jnp.int32),
         10104 + (jnp.arange(npad, dtype=jnp.int32) & 7)]
    ).reshape(16, _NCH, _K)
    b_edge2 = b_edge.reshape(1, _D)
    b_ih2 = b_ih.reshape(1, 3 * _D)
    b_hh2 = b_hh.reshape(1, 3 * _D)
    b_fc2 = b_fc.reshape(1, _NCLASS)

    m1 = _edge_mm(x, W_edge, b_edge2)
    p1 = _sc_scatter(m1, src, dst)
    h1, m2 = _gru_edge(p1, x, W_ih, W_hh, b_ih2, b_hh2, W_edge, b_edge2)
    p2 = _sc_scatter(m2, src, dst)
    logits = _gru_fc(p2, h1, W_ih, W_hh, b_ih2, b_hh2, W_fc, b_fc2)
    return logits


# speedup vs baseline: 1.4425x; 1.0982x over previous
import functools

import jax
import jax.numpy as jnp
from jax import lax
from jax.experimental import pallas as pl
from jax.experimental.pallas import tpu as pltpu
from jax.experimental.pallas import tpu_sc as plsc

_N = 10000
_E = 320000
_D = 128
_DH = 64
_NCLASS = 16

_NCH = 160
_K = 125
_NPAD = 10112
_ROWS_PER_TILE = _NPAD // 16

_BN = 1000


def _sc_scatter_build():
    mesh = plsc.VectorSubcoreMesh(core_axis_name="c", subcore_axis_name="s")

    @functools.partial(
        pl.kernel,
        out_type=jax.ShapeDtypeStruct((2, _NPAD, _DH), jnp.float32),
        mesh=mesh,
        scratch_types=[
            pltpu.VMEM((_NCH, _K), jnp.int32),
            pltpu.VMEM((_NCH, _K), jnp.int32),
            pltpu.VMEM((_K, _DH), jnp.float32),
            pltpu.VMEM((_ROWS_PER_TILE, _DH), jnp.float32),
            pltpu.VMEM_SHARED((_NPAD, _DH), jnp.float32),
            pltpu.SemaphoreType.DMA,
        ],
        compiler_params=pltpu.CompilerParams(use_tc_tiling_on_sc=False),
    )
    def sc_scatter(m2_hbm, src_hbm, dst_hbm, out_hbm,
                   src_v, dst_v, rows_v, z_v, acc_sh, gsem):
        c = lax.axis_index("c")
        s = lax.axis_index("s")

        pltpu.sync_copy(src_hbm.at[s], src_v)
        pltpu.sync_copy(dst_hbm.at[s], dst_v)

        zero = jnp.zeros((16,), jnp.float32)

        def zrow(r, carry):
            for cc in range(_DH // 16):
                z_v[r, pl.ds(cc * 16, 16)] = zero
            return carry

        lax.fori_loop(0, _ROWS_PER_TILE, zrow, 0)

        row0 = s * _ROWS_PER_TILE
        pltpu.sync_copy(z_v, acc_sh.at[pl.ds(row0, _ROWS_PER_TILE)])
        plsc.subcore_barrier()

        m_ref = m2_hbm.at[c]

        def body(j, carry):
            pltpu.async_copy(m_ref.at[src_v.at[j]], rows_v, gsem).wait()
            pltpu.sync_copy(rows_v, acc_sh.at[dst_v.at[j]], add=True)
            return carry

        lax.fori_loop(0, _NCH, body, 0)

        plsc.subcore_barrier()

        pltpu.sync_copy(acc_sh.at[pl.ds(row0, _ROWS_PER_TILE)],
                        out_hbm.at[c, pl.ds(row0, _ROWS_PER_TILE)])

    return sc_scatter


_sc_scatter = _sc_scatter_build()


def _edge_mm_body(h_ref, W_ref, b_ref, m2_ref):
    m = (
        jnp.dot(h_ref[...], W_ref[...], preferred_element_type=jnp.float32)
        + b_ref[...]
    )
    m2_ref[0] = m[:, :_DH]
    m2_ref[1] = m[:, _DH:]


def _gru(p_ref, h_ref, W_ih_ref, W_hh_ref, b_ih_ref, b_hh_ref):
    a_lo = p_ref[0]
    a_hi = p_ref[1]
    gi = (
        jnp.dot(a_lo, W_ih_ref[...][:_DH, :], preferred_element_type=jnp.float32)
        + jnp.dot(a_hi, W_ih_ref[...][_DH:, :], preferred_element_type=jnp.float32)
        + b_ih_ref[...]
    )
    h = h_ref[...]
    gh = jnp.dot(h, W_hh_ref[...], preferred_element_type=jnp.float32) + b_hh_ref[...]
    r = jax.nn.sigmoid(gi[:, :_D] + gh[:, :_D])
    z = jax.nn.sigmoid(gi[:, _D:2 * _D] + gh[:, _D:2 * _D])
    n = jnp.tanh(gi[:, 2 * _D:] + r * gh[:, 2 * _D:])
    return (1.0 - z) * n + z * h


def _gru_edge_body(p_ref, h_ref, W_ih_ref, W_hh_ref, b_ih_ref, b_hh_ref,
                   W_edge_ref, b_edge_ref, hn_ref, m2_ref):
    hn = _gru(p_ref, h_ref, W_ih_ref, W_hh_ref, b_ih_ref, b_hh_ref)
    hn_ref[...] = hn
    m = (
        jnp.dot(hn, W_edge_ref[...], preferred_element_type=jnp.float32)
        + b_edge_ref[...]
    )
    m2_ref[0] = m[:, :_DH]
    m2_ref[1] = m[:, _DH:]


def _gru_fc_body(p_ref, h_ref, W_ih_ref, W_hh_ref, b_ih_ref, b_hh_ref,
                 W_fc_ref, b_fc_ref, out_ref):
    hn = _gru(p_ref, h_ref, W_ih_ref, W_hh_ref, b_ih_ref, b_hh_ref)
    e = jnp.where(hn > 0, hn, jnp.exp(jnp.minimum(hn, 0.0)) - 1.0)
    out_ref[...] = (
        jnp.dot(e, W_fc_ref[...], preferred_element_type=jnp.float32)
        + b_fc_ref[...]
    )


def _full(shape):
    return pl.BlockSpec(shape, lambda i: tuple(0 for _ in shape))


_GRID = _N // _BN

_edge_mm = pl.pallas_call(
    _edge_mm_body,
    grid=(_GRID,),
    in_specs=[
        pl.BlockSpec((_BN, _D), lambda i: (i, 0)),
        _full((_D, _D)),
        _full((1, _D)),
    ],
    out_specs=pl.BlockSpec((2, _BN, _DH), lambda i: (0, i, 0)),
    out_shape=jax.ShapeDtypeStruct((2, _N, _DH), jnp.float32),
)

_gru_edge = pl.pallas_call(
    _gru_edge_body,
    grid=(_GRID,),
    in_specs=[
        pl.BlockSpec((2, _BN, _DH), lambda i: (0, i, 0)),
        pl.BlockSpec((_BN, _D), lambda i: (i, 0)),
        _full((_D, 3 * _D)),
        _full((_D, 3 * _D)),
        _full((1, 3 * _D)),
        _full((1, 3 * _D)),
        _full((_D, _D)),
        _full((1, _D)),
    ],
    out_specs=[
        pl.BlockSpec((_BN, _D), lambda i: (i, 0)),
        pl.BlockSpec((2, _BN, _DH), lambda i: (0, i, 0)),
    ],
    out_shape=[
        jax.ShapeDtypeStruct((_N, _D), jnp.float32),
        jax.ShapeDtypeStruct((2, _N, _DH), jnp.float32),
    ],
)

_gru_fc = pl.pallas_call(
    _gru_fc_body,
    grid=(_GRID,),
    in_specs=[
        pl.BlockSpec((2, _BN, _DH), lambda i: (0, i, 0)),
        pl.BlockSpec((_BN, _D), lambda i: (i, 0)),
        _full((_D, 3 * _D)),
        _full((_D, 3 * _D)),
        _full((1, 3 * _D)),
        _full((1, 3 * _D)),
        _full((_D, _NCLASS)),
        _full((1, _NCLASS)),
    ],
    out_specs=pl.BlockSpec((_BN, _NCLASS), lambda i: (i, 0)),
    out_shape=jax.ShapeDtypeStruct((_N, _NCLASS), jnp.float32),
)


def kernel(x, edge_index, W_edge, b_edge, W_ih, W_hh, b_ih, b_hh, W_fc, b_fc):
    src = edge_index[0].astype(jnp.int32).reshape(16, _NCH, _K)
    dst = edge_index[1].astype(jnp.int32).reshape(16, _NCH, _K)
    b_edge2 = b_edge.reshape(1, _D)
    b_ih2 = b_ih.reshape(1, 3 * _D)
    b_hh2 = b_hh.reshape(1, 3 * _D)
    b_fc2 = b_fc.reshape(1, _NCLASS)

    m1 = _edge_mm(x, W_edge, b_edge2)
    p1 = _sc_scatter(m1, src, dst)
    h1, m2 = _gru_edge(p1, x, W_ih, W_hh, b_ih2, b_hh2, W_edge, b_edge2)
    p2 = _sc_scatter(m2, src, dst)
    logits = _gru_fc(p2, h1, W_ih, W_hh, b_ih2, b_hh2, W_fc, b_fc2)
    return logits
